# baseline (speedup 1.0000x reference)
import functools

import jax
import jax.numpy as jnp
from jax import lax
from jax.experimental import pallas as pl
from jax.experimental.pallas import tpu as pltpu
from jax.experimental.pallas import tpu_sc as plsc

B = 16384
C = 1001
NADV = 1000
S = 1.0 / NADV

NC, NS, L = 2, 16, 16
NW = NC * NS
PW = B // NW
CH = 32
NCH = PW // CH
NG = CH // L


def _make_sc():
    mesh = plsc.VectorSubcoreMesh(core_axis_name="c", subcore_axis_name="s")

    @functools.partial(
        pl.kernel,
        out_type=jax.ShapeDtypeStruct((B,), jnp.float32),
        mesh=mesh,
        compiler_params=pltpu.CompilerParams(use_tc_tiling_on_sc=False, needs_layout_passes=False),
        scratch_types=[
            pltpu.VMEM((2, CH * NADV), jnp.float32),
            pltpu.VMEM((PW,), jnp.int32),
            pltpu.VMEM((PW,), jnp.float32),
            pltpu.SemaphoreType.DMA((2,)),
            pltpu.SemaphoreType.DMA,
        ],
    )
    def sc_duelqa(x_hbm, a_hbm, out_hbm, xv, av, ov, sems, asem):
        wid = lax.axis_index("s") * NC + lax.axis_index("c")
        base = wid * PW
        pltpu.async_copy(a_hbm.at[pl.ds(base, PW)], av, asem).wait()
        lane = lax.iota(jnp.int32, L)
        zero16 = jnp.zeros((L,), jnp.float32)

        def row_cp(c, b, r):
            return pltpu.make_async_copy(
                x_hbm.at[base + c * CH + r, pl.ds(0, NADV)],
                xv.at[b, pl.ds(r * NADV, NADV)],
                sems.at[b],
            )

        def start_chunk(c, b):
            for r in range(CH):
                row_cp(c, b, r).start()

        def wait_chunk(c, b):
            for r in range(CH):
                row_cp(c, b, r).wait()

        start_chunk(0, 0)
        start_chunk(1, 1)

        def chunk_body(c, carry):
            b = lax.rem(c, 2)
            wait_chunk(c, b)
            xb = xv.at[b]
            for g in range(NG):
                idx0 = (g * L + lane) * NADV

                def body(j, acc):
                    return acc + plsc.load_gather(xb, [idx0 + j])

                t = lax.fori_loop(0, NADV, body, zero16, unroll=16)
                off = c * CH + g * L
                a16 = av[pl.ds(off, L)]
                gv = plsc.load_gather(xb, [idx0 + a16])
                ov[pl.ds(off, L)] = gv - t * jnp.float32(S)

            @pl.when(c + 2 < NCH)
            def _():
                start_chunk(c + 2, b)

            return carry

        lax.fori_loop(0, NCH, chunk_body, 0)
        pltpu.sync_copy(ov, out_hbm.at[pl.ds(base, PW)])

    return sc_duelqa


_SC = _make_sc()


def kernel(x, a):
    a32 = a.reshape(-1).astype(jnp.int32)
    partial = _SC(x, a32)
    return (partial + x[:, NADV])[:, None]


# speedup vs baseline: 1.3569x; 1.3569x over previous
import functools

import jax
import jax.numpy as jnp
from jax import lax
from jax.experimental import pallas as pl
from jax.experimental.pallas import tpu as pltpu
from jax.experimental.pallas import tpu_sc as plsc

B = 16384
C = 1001
NADV = 1000
S = 1.0 / NADV

NC, NS, L = 2, 16, 16
NW = NC * NS
PW = B // NW
CH = 32
NCH = PW // CH
NG = CH // L
NFULL = 62


def _make_sc():
    mesh = plsc.VectorSubcoreMesh(core_axis_name="c", subcore_axis_name="s")

    @functools.partial(
        pl.kernel,
        out_type=jax.ShapeDtypeStruct((B,), jnp.float32),
        mesh=mesh,
        compiler_params=pltpu.CompilerParams(needs_layout_passes=False),
        scratch_types=[
            pltpu.VMEM((2, CH, C), jnp.float32),
            pltpu.VMEM((PW,), jnp.int32),
            pltpu.VMEM((PW,), jnp.float32),
            pltpu.SemaphoreType.DMA((2,)),
            pltpu.SemaphoreType.DMA,
        ],
    )
    def sc_duelqa(x_hbm, a_hbm, out_hbm, xv, av, ov, sems, asem):
        wid = lax.axis_index("s") * NC + lax.axis_index("c")
        base = wid * PW
        pltpu.async_copy(a_hbm.at[pl.ds(base, PW)], av, asem).wait()
        lane = lax.iota(jnp.int32, L)
        zero16 = jnp.zeros((L,), jnp.float32)
        tailm = (lane >= 8).astype(jnp.float32)
        onehot = [(lane == r).astype(jnp.float32) for r in range(L)]

        def cp(c, b):
            return pltpu.make_async_copy(
                x_hbm.at[pl.ds(base + c * CH, CH), :],
                xv.at[b],
                sems.at[b],
            )

        cp(0, 0).start()
        cp(1, 1).start()

        def chunk_body(c, carry):
            b = lax.rem(c, 2)
            cp(c, b).wait()
            for g in range(NG):
                tvec = zero16
                for r in range(L):
                    row = g * L + r
                    accs = [zero16, zero16, zero16, zero16]
                    for j in range(NFULL):
                        accs[j % 4] = accs[j % 4] + xv[b, row, pl.ds(j * L, L)]
                    tail = xv[b, row, pl.ds(984, L)] * tailm
                    acc = (accs[0] + accs[1]) + (accs[2] + accs[3]) + tail
                    tvec = tvec + jnp.sum(acc) * onehot[r]
                off = c * CH + g * L
                a16 = av[pl.ds(off, L)]
                gv = plsc.load_gather(xv.at[b], [g * L + lane, a16])
                ov[pl.ds(off, L)] = gv - tvec * jnp.float32(S)

            @pl.when(c + 2 < NCH)
            def _():
                cp(c + 2, b).start()

            return carry

        lax.fori_loop(0, NCH, chunk_body, 0)
        pltpu.sync_copy(ov, out_hbm.at[pl.ds(base, PW)])

    return sc_duelqa


_SC = _make_sc()


def kernel(x, a):
    a32 = a.reshape(-1).astype(jnp.int32)
    partial = _SC(x, a32)
    return (partial + x[:, NADV])[:, None]
